# Initial kernel scaffold; baseline (speedup 1.0000x reference)
import jax
import jax.numpy as jnp
from jax.experimental import pallas as pl


def kernel(x, embedding, W_lin, att_i, att_j, att_em_i, att_em_j, gnn_bias, W1, b1, W2, b2):
    raise NotImplementedError("write your pallas kernel here")


# speedup vs baseline: 12.3409x; 12.3409x over previous
import functools
import math

import jax
import jax.numpy as jnp
from jax.experimental import pallas as pl

_EPS = 1e-5
_TOPK = 16
_NEG = -jnp.inf


def _prep_kernel(x_ref, wlin_ref, emb_ref, ai_ref, aj_ref, aei_ref, aej_ref,
                 xl_ref, adst_ref, asrc_ref, inrm_ref):
    emb = emb_ref[...]
    inrm_ref[0, :] = jax.lax.rsqrt(jnp.sum(emb * emb, axis=1))
    e_i = jnp.sum(emb * aei_ref[0, :][None, :], axis=1)
    e_j = jnp.sum(emb * aej_ref[0, :][None, :], axis=1)
    wl = wlin_ref[...]
    nb = x_ref.shape[0]
    for b in range(nb):
        xlb = jax.lax.dot(x_ref[b], wl, precision=jax.lax.Precision.HIGHEST)
        xl_ref[b] = xlb
        adst_ref[b, :] = jnp.sum(xlb * ai_ref[0, :][None, :], axis=1) + e_i
        asrc_ref[b, :] = jnp.sum(xlb * aj_ref[0, :][None, :], axis=1) + e_j


def _main_kernel(embr_ref, embf_ref, inrm_ref, xl_ref, adst_ref, asrc_ref,
                 bias_ref, w1_ref, b1_ref, w2_ref, b2_ref, out_ref,
                 *, blk_r, n_nodes):
    g = pl.program_id(0)
    embr = embr_ref[...]
    t = jax.lax.dot_general(embr, embf_ref[...],
                            (((1,), (1,)), ((), ())),
                            precision=jax.lax.Precision.HIGHEST)
    t = t * inrm_ref[0, :][None, :]
    colid = jax.lax.broadcasted_iota(jnp.int32, t.shape, 1)
    for _ in range(_TOPK):
        m = jnp.max(t, axis=1, keepdims=True)
        cand = jnp.where(t >= m, colid, n_nodes)
        j = jnp.min(cand, axis=1, keepdims=True)
        t = jnp.where(colid == j, _NEG, t)
    rowg = g * blk_r + jax.lax.broadcasted_iota(jnp.int32, t.shape, 0)
    sel = jnp.logical_or(t == _NEG, colid == rowg)
    c = 1.0 / math.sqrt(1.0 + _EPS)
    bias = bias_ref[0, :]
    nb = adst_ref.shape[0]
    for b in range(nb):
        alpha = adst_ref[b, :][:, None] + asrc_ref[b, :][None, :]
        alpha = jnp.where(alpha > 0, alpha, 0.2 * alpha)
        am = jnp.where(sel, alpha, _NEG)
        mx = jnp.max(am, axis=1, keepdims=True)
        ex = jnp.exp(am - mx)
        den = jnp.sum(ex, axis=1, keepdims=True) + 1e-16
        w = ex / den
        agg = jax.lax.dot(w, xl_ref[b],
                          precision=jax.lax.Precision.HIGHEST)
        gagg = jnp.maximum((agg + bias[None, :]) * c, 0.0)
        h = jnp.maximum(gagg * embr * c, 0.0)
        o = jax.lax.dot(h, w1_ref[...],
                        precision=jax.lax.Precision.HIGHEST) + b1_ref[0, :][None, :]
        o = jax.lax.dot(o, w2_ref[...],
                        precision=jax.lax.Precision.HIGHEST) + b2_ref[0, :][None, :]
        out_ref[b, :] = jnp.maximum(o[:, 0] * c, 0.0)


def kernel(x, embedding, W_lin, att_i, att_j, att_em_i, att_em_j, gnn_bias,
           W1, b1, W2, b2):
    B, D, T = x.shape
    DIM = embedding.shape[1]
    ai = att_i.reshape(1, DIM)
    aj = att_j.reshape(1, DIM)
    aei = att_em_i.reshape(1, DIM)
    aej = att_em_j.reshape(1, DIM)
    bias = gnn_bias.reshape(1, DIM)
    b1r = b1.reshape(1, -1)
    b2r = b2.reshape(1, -1)

    r0 = 512
    nb0 = (D + r0 - 1) // r0
    xl, a_dst, a_src, inrm = pl.pallas_call(
        _prep_kernel,
        grid=(nb0,),
        in_specs=[
            pl.BlockSpec((B, r0, T), lambda i: (0, i, 0)),
            pl.BlockSpec((T, DIM), lambda i: (0, 0)),
            pl.BlockSpec((r0, DIM), lambda i: (i, 0)),
            pl.BlockSpec((1, DIM), lambda i: (0, 0)),
            pl.BlockSpec((1, DIM), lambda i: (0, 0)),
            pl.BlockSpec((1, DIM), lambda i: (0, 0)),
            pl.BlockSpec((1, DIM), lambda i: (0, 0)),
        ],
        out_specs=[
            pl.BlockSpec((B, r0, DIM), lambda i: (0, i, 0)),
            pl.BlockSpec((B, r0), lambda i: (0, i)),
            pl.BlockSpec((B, r0), lambda i: (0, i)),
            pl.BlockSpec((1, r0), lambda i: (0, i)),
        ],
        out_shape=[
            jax.ShapeDtypeStruct((B, D, DIM), jnp.float32),
            jax.ShapeDtypeStruct((B, D), jnp.float32),
            jax.ShapeDtypeStruct((B, D), jnp.float32),
            jax.ShapeDtypeStruct((1, D), jnp.float32),
        ],
    )(x, W_lin, embedding, ai, aj, aei, aej)

    r1 = 128
    nb1 = (D + r1 - 1) // r1
    out = pl.pallas_call(
        functools.partial(_main_kernel, blk_r=r1, n_nodes=D),
        grid=(nb1,),
        in_specs=[
            pl.BlockSpec((r1, DIM), lambda i: (i, 0)),
            pl.BlockSpec((D, DIM), lambda i: (0, 0)),
            pl.BlockSpec((1, D), lambda i: (0, 0)),
            pl.BlockSpec((B, D, DIM), lambda i: (0, 0, 0)),
            pl.BlockSpec((B, r1), lambda i: (0, i)),
            pl.BlockSpec((B, D), lambda i: (0, 0)),
            pl.BlockSpec((1, DIM), lambda i: (0, 0)),
            pl.BlockSpec((DIM, W1.shape[1]), lambda i: (0, 0)),
            pl.BlockSpec((1, b1.shape[0]), lambda i: (0, 0)),
            pl.BlockSpec((W2.shape[0], 1), lambda i: (0, 0)),
            pl.BlockSpec((1, 1), lambda i: (0, 0)),
        ],
        out_specs=pl.BlockSpec((B, r1), lambda i: (0, i)),
        out_shape=jax.ShapeDtypeStruct((B, D), jnp.float32),
    )(embedding, embedding, inrm, xl, a_dst, a_src, bias, W1, b1r, W2, b2r)
    return out


# speedup vs baseline: 16.8596x; 1.3662x over previous
import functools
import math

import jax
import jax.numpy as jnp
from jax.experimental import pallas as pl

_EPS = 1e-5
_TOPK = 16
_NEG = -jnp.inf


def _prep_kernel(x_ref, wlin_ref, emb_ref, ai_ref, aj_ref, aei_ref, aej_ref,
                 xl_ref, adst_ref, asrc_ref, inrm_ref):
    emb = emb_ref[...]
    inrm_ref[0, :] = jax.lax.rsqrt(jnp.sum(emb * emb, axis=1))
    e_i = jnp.sum(emb * aei_ref[0, :][None, :], axis=1)
    e_j = jnp.sum(emb * aej_ref[0, :][None, :], axis=1)
    wl = wlin_ref[...]
    nb = x_ref.shape[0]
    for b in range(nb):
        xlb = jax.lax.dot(x_ref[b], wl, precision=jax.lax.Precision.HIGHEST)
        xl_ref[b] = xlb
        adst_ref[b, :] = jnp.sum(xlb * ai_ref[0, :][None, :], axis=1) + e_i
        asrc_ref[b, :] = jnp.sum(xlb * aj_ref[0, :][None, :], axis=1) + e_j


def _main_kernel(embr_ref, embf_ref, inrm_ref, xl_ref, adst_ref, asrc_ref,
                 bias_ref, w1_ref, b1_ref, w2_ref, b2_ref, out_ref,
                 *, blk_r, n_nodes):
    g = pl.program_id(0)
    embr = embr_ref[...]
    t = jax.lax.dot_general(embr, embf_ref[...],
                            (((1,), (1,)), ((), ())),
                            precision=jax.lax.Precision.HIGHEST)
    t = t * inrm_ref[0, :][None, :]
    colid = jax.lax.broadcasted_iota(jnp.int32, t.shape, 1)
    for _ in range(_TOPK):
        m = jnp.max(t, axis=1, keepdims=True)
        cand = jnp.where(t >= m, colid, n_nodes)
        j = jnp.min(cand, axis=1, keepdims=True)
        t = jnp.where(colid == j, _NEG, t)
    rowg = g * blk_r + jax.lax.broadcasted_iota(jnp.int32, t.shape, 0)
    sel = jnp.logical_or(t == _NEG, colid == rowg)
    c = 1.0 / math.sqrt(1.0 + _EPS)
    bias = bias_ref[0, :]
    nb = adst_ref.shape[0]
    for b in range(nb):
        alpha = adst_ref[b, :][:, None] + asrc_ref[b, :][None, :]
        alpha = jnp.where(alpha > 0, alpha, 0.2 * alpha)
        am = jnp.where(sel, alpha, _NEG)
        mx = jnp.max(am, axis=1, keepdims=True)
        ex = jnp.exp(am - mx)
        den = jnp.sum(ex, axis=1, keepdims=True) + 1e-16
        w = ex / den
        agg = jax.lax.dot(w, xl_ref[b])
        gagg = jnp.maximum((agg + bias[None, :]) * c, 0.0)
        h = jnp.maximum(gagg * embr * c, 0.0)
        o = jax.lax.dot(h, w1_ref[...]) + b1_ref[0, :][None, :]
        o = jax.lax.dot(o, w2_ref[...]) + b2_ref[0, :][None, :]
        out_ref[b, :] = jnp.maximum(o[:, 0] * c, 0.0)


def kernel(x, embedding, W_lin, att_i, att_j, att_em_i, att_em_j, gnn_bias,
           W1, b1, W2, b2):
    B, D, T = x.shape
    DIM = embedding.shape[1]
    ai = att_i.reshape(1, DIM)
    aj = att_j.reshape(1, DIM)
    aei = att_em_i.reshape(1, DIM)
    aej = att_em_j.reshape(1, DIM)
    bias = gnn_bias.reshape(1, DIM)
    b1r = b1.reshape(1, -1)
    b2r = b2.reshape(1, -1)

    r0 = 512
    nb0 = (D + r0 - 1) // r0
    xl, a_dst, a_src, inrm = pl.pallas_call(
        _prep_kernel,
        grid=(nb0,),
        in_specs=[
            pl.BlockSpec((B, r0, T), lambda i: (0, i, 0)),
            pl.BlockSpec((T, DIM), lambda i: (0, 0)),
            pl.BlockSpec((r0, DIM), lambda i: (i, 0)),
            pl.BlockSpec((1, DIM), lambda i: (0, 0)),
            pl.BlockSpec((1, DIM), lambda i: (0, 0)),
            pl.BlockSpec((1, DIM), lambda i: (0, 0)),
            pl.BlockSpec((1, DIM), lambda i: (0, 0)),
        ],
        out_specs=[
            pl.BlockSpec((B, r0, DIM), lambda i: (0, i, 0)),
            pl.BlockSpec((B, r0), lambda i: (0, i)),
            pl.BlockSpec((B, r0), lambda i: (0, i)),
            pl.BlockSpec((1, r0), lambda i: (0, i)),
        ],
        out_shape=[
            jax.ShapeDtypeStruct((B, D, DIM), jnp.float32),
            jax.ShapeDtypeStruct((B, D), jnp.float32),
            jax.ShapeDtypeStruct((B, D), jnp.float32),
            jax.ShapeDtypeStruct((1, D), jnp.float32),
        ],
    )(x, W_lin, embedding, ai, aj, aei, aej)

    r1 = 128
    nb1 = (D + r1 - 1) // r1
    out = pl.pallas_call(
        functools.partial(_main_kernel, blk_r=r1, n_nodes=D),
        grid=(nb1,),
        in_specs=[
            pl.BlockSpec((r1, DIM), lambda i: (i, 0)),
            pl.BlockSpec((D, DIM), lambda i: (0, 0)),
            pl.BlockSpec((1, D), lambda i: (0, 0)),
            pl.BlockSpec((B, D, DIM), lambda i: (0, 0, 0)),
            pl.BlockSpec((B, r1), lambda i: (0, i)),
            pl.BlockSpec((B, D), lambda i: (0, 0)),
            pl.BlockSpec((1, DIM), lambda i: (0, 0)),
            pl.BlockSpec((DIM, W1.shape[1]), lambda i: (0, 0)),
            pl.BlockSpec((1, b1.shape[0]), lambda i: (0, 0)),
            pl.BlockSpec((W2.shape[0], 1), lambda i: (0, 0)),
            pl.BlockSpec((1, 1), lambda i: (0, 0)),
        ],
        out_specs=pl.BlockSpec((B, r1), lambda i: (0, i)),
        out_shape=jax.ShapeDtypeStruct((B, D), jnp.float32),
    )(embedding, embedding, inrm, xl, a_dst, a_src, bias, W1, b1r, W2, b2r)
    return out


# speedup vs baseline: 16.8717x; 1.0007x over previous
import functools
import math

import jax
import jax.numpy as jnp
from jax.experimental import pallas as pl

_EPS = 1e-5
_TOPK = 16
_NEG = -jnp.inf


def _prep_kernel(x_ref, wlin_ref, emb_ref, ai_ref, aj_ref, aei_ref, aej_ref,
                 xl_ref, adst_ref, asrc_ref, inrm_ref):
    emb = emb_ref[...]
    inrm_ref[0, :] = 1.0 / jnp.sqrt(jnp.sum(emb * emb, axis=1))
    e_i = jnp.sum(emb * aei_ref[0, :][None, :], axis=1)
    e_j = jnp.sum(emb * aej_ref[0, :][None, :], axis=1)
    wl = wlin_ref[...]
    nb = x_ref.shape[0]
    for b in range(nb):
        xlb = jax.lax.dot(x_ref[b], wl, precision=jax.lax.Precision.HIGHEST)
        xl_ref[b] = xlb
        adst_ref[b, :] = jnp.sum(xlb * ai_ref[0, :][None, :], axis=1) + e_i
        asrc_ref[b, :] = jnp.sum(xlb * aj_ref[0, :][None, :], axis=1) + e_j


def _main_kernel(embr_ref, embf_ref, inrm_ref, xl_ref, adst_ref, asrc_ref,
                 bias_ref, w1_ref, b1_ref, w2_ref, b2_ref, out_ref,
                 *, blk_r, n_nodes):
    g = pl.program_id(0)
    embr = embr_ref[...]
    t = jax.lax.dot_general(embr, embf_ref[...],
                            (((1,), (1,)), ((), ())),
                            precision=jax.lax.Precision.HIGHEST)
    t = t * inrm_ref[0, :][None, :]
    colid = jax.lax.broadcasted_iota(jnp.int32, t.shape, 1)
    for _ in range(_TOPK):
        m = jnp.max(t, axis=1, keepdims=True)
        cand = jnp.where(t >= m, colid, n_nodes)
        j = jnp.min(cand, axis=1, keepdims=True)
        t = jnp.where(colid == j, _NEG, t)
    rowg = g * blk_r + jax.lax.broadcasted_iota(jnp.int32, t.shape, 0)
    sel = jnp.logical_or(t == _NEG, colid == rowg)
    c = 1.0 / math.sqrt(1.0 + _EPS)
    bias = bias_ref[0, :]
    nb = adst_ref.shape[0]
    for b in range(nb):
        alpha = adst_ref[b, :][:, None] + asrc_ref[b, :][None, :]
        alpha = jnp.where(alpha > 0, alpha, 0.2 * alpha)
        am = jnp.where(sel, alpha, _NEG)
        mx = jnp.max(am, axis=1, keepdims=True)
        ex = jnp.exp(am - mx)
        den = jnp.sum(ex, axis=1, keepdims=True) + 1e-16
        w = ex / den
        agg = jax.lax.dot(w, xl_ref[b])
        gagg = jnp.maximum((agg + bias[None, :]) * c, 0.0)
        h = jnp.maximum(gagg * embr * c, 0.0)
        o = jax.lax.dot(h, w1_ref[...]) + b1_ref[0, :][None, :]
        o = jax.lax.dot(o, w2_ref[...]) + b2_ref[0, :][None, :]
        out_ref[b, :] = jnp.maximum(o[:, 0] * c, 0.0)


def kernel(x, embedding, W_lin, att_i, att_j, att_em_i, att_em_j, gnn_bias,
           W1, b1, W2, b2):
    B, D, T = x.shape
    DIM = embedding.shape[1]
    ai = att_i.reshape(1, DIM)
    aj = att_j.reshape(1, DIM)
    aei = att_em_i.reshape(1, DIM)
    aej = att_em_j.reshape(1, DIM)
    bias = gnn_bias.reshape(1, DIM)
    b1r = b1.reshape(1, -1)
    b2r = b2.reshape(1, -1)

    r0 = 512
    nb0 = (D + r0 - 1) // r0
    xl, a_dst, a_src, inrm = pl.pallas_call(
        _prep_kernel,
        grid=(nb0,),
        in_specs=[
            pl.BlockSpec((B, r0, T), lambda i: (0, i, 0)),
            pl.BlockSpec((T, DIM), lambda i: (0, 0)),
            pl.BlockSpec((r0, DIM), lambda i: (i, 0)),
            pl.BlockSpec((1, DIM), lambda i: (0, 0)),
            pl.BlockSpec((1, DIM), lambda i: (0, 0)),
            pl.BlockSpec((1, DIM), lambda i: (0, 0)),
            pl.BlockSpec((1, DIM), lambda i: (0, 0)),
        ],
        out_specs=[
            pl.BlockSpec((B, r0, DIM), lambda i: (0, i, 0)),
            pl.BlockSpec((B, r0), lambda i: (0, i)),
            pl.BlockSpec((B, r0), lambda i: (0, i)),
            pl.BlockSpec((1, r0), lambda i: (0, i)),
        ],
        out_shape=[
            jax.ShapeDtypeStruct((B, D, DIM), jnp.float32),
            jax.ShapeDtypeStruct((B, D), jnp.float32),
            jax.ShapeDtypeStruct((B, D), jnp.float32),
            jax.ShapeDtypeStruct((1, D), jnp.float32),
        ],
    )(x, W_lin, embedding, ai, aj, aei, aej)

    r1 = 128
    nb1 = (D + r1 - 1) // r1
    out = pl.pallas_call(
        functools.partial(_main_kernel, blk_r=r1, n_nodes=D),
        grid=(nb1,),
        in_specs=[
            pl.BlockSpec((r1, DIM), lambda i: (i, 0)),
            pl.BlockSpec((D, DIM), lambda i: (0, 0)),
            pl.BlockSpec((1, D), lambda i: (0, 0)),
            pl.BlockSpec((B, D, DIM), lambda i: (0, 0, 0)),
            pl.BlockSpec((B, r1), lambda i: (0, i)),
            pl.BlockSpec((B, D), lambda i: (0, 0)),
            pl.BlockSpec((1, DIM), lambda i: (0, 0)),
            pl.BlockSpec((DIM, W1.shape[1]), lambda i: (0, 0)),
            pl.BlockSpec((1, b1.shape[0]), lambda i: (0, 0)),
            pl.BlockSpec((W2.shape[0], 1), lambda i: (0, 0)),
            pl.BlockSpec((1, 1), lambda i: (0, 0)),
        ],
        out_specs=pl.BlockSpec((B, r1), lambda i: (0, i)),
        out_shape=jax.ShapeDtypeStruct((B, D), jnp.float32),
    )(embedding, embedding, inrm, xl, a_dst, a_src, bias, W1, b1r, W2, b2r)
    return out
